# baseline (speedup 1.0000x reference)
import jax
import jax.numpy as jnp
from jax import lax
from jax.experimental import pallas as pl
from jax.experimental.pallas import tpu as pltpu
from jax.experimental.pallas import tpu_sc as plsc

B, N, DPTS = 8, 2048, 64
NPOINT, RADIUS, NSAMPLE = 512, 0.2, 32
FW = 128
NROWS = B * NPOINT
P = NROWS * NSAMPLE
NW = 32
RPW = NROWS // NW
RB = 4096
GB = RB // NSAMPLE
NBLK = P // RB


NGF = 4
HB = B // NGF


def _fps_body(a_ref, idx_ref, cx_ref, cy_ref, cz_ref):
    a = a_ref[...]
    lane = jax.lax.broadcasted_iota(jnp.int32, (HB, N), 1)
    lane3 = jax.lax.broadcasted_iota(jnp.int32, (3 * HB, N), 1)
    out_lane = jax.lax.broadcasted_iota(jnp.int32, (HB, NPOINT), 1)
    slabs = [a[3 * HB * g:3 * HB * (g + 1)] for g in range(NGF)]

    def step_grp(slab, st, i):
        dist, far, cents, ox, oy, oz = st
        far3 = jnp.concatenate([far, far, far], axis=0)
        oh = lane3 == far3
        cvec = jnp.sum(jnp.where(oh, slab, 0.0), axis=1, keepdims=True)
        sel = out_lane == i
        cents = jnp.where(sel, far, cents)
        ox = jnp.where(sel, cvec[0:HB], ox)
        oy = jnp.where(sel, cvec[HB:2 * HB], oy)
        oz = jnp.where(sel, cvec[2 * HB:3 * HB], oz)
        df = slab - cvec
        sq = df * df
        d = sq[0:HB] + sq[HB:2 * HB]
        d = d + sq[2 * HB:3 * HB]
        dist = jnp.minimum(dist, d)
        m = jnp.max(dist, axis=1, keepdims=True)
        far = jnp.min(jnp.where(dist == m, lane, N), axis=1, keepdims=True)
        return (dist, far, cents, ox, oy, oz)

    def body(i, sts):
        return tuple(step_grp(slabs[g], sts[g], i) for g in range(NGF))

    def init(slab):
        dist0 = jnp.full((HB, N), 1e10, dtype=jnp.float32) + slab[0:HB] * 0.0
        far0 = jnp.min(lane, axis=1, keepdims=True)
        ox0 = slab[0:HB, :NPOINT] * 0.0
        return (dist0, far0, ox0.astype(jnp.int32), ox0, ox0, ox0)

    sts = jax.lax.fori_loop(0, NPOINT, body,
                            tuple(init(slabs[g]) for g in range(NGF)))
    idx_ref[...] = jnp.concatenate([st[2] for st in sts], axis=0)
    cx_ref[...] = jnp.concatenate([st[3] for st in sts], axis=0)
    cy_ref[...] = jnp.concatenate([st[4] for st in sts], axis=0)
    cz_ref[...] = jnp.concatenate([st[5] for st in sts], axis=0)


def _run_fps(xyz):
    a = xyz.reshape(NGF, HB, 3, N).transpose(0, 2, 1, 3).reshape(3 * B, N)
    out_shape = [
        jax.ShapeDtypeStruct((B, NPOINT), jnp.int32),
        jax.ShapeDtypeStruct((B, NPOINT), jnp.float32),
        jax.ShapeDtypeStruct((B, NPOINT), jnp.float32),
        jax.ShapeDtypeStruct((B, NPOINT), jnp.float32),
    ]
    return pl.pallas_call(_fps_body, out_shape=out_shape)(a)


NCH = N // 16


def _mask_body(c_ref, p_ref, f_ref, slo_ref, shi_ref, ft_ref,
               sg_ref, idx_ref, gff_ref, gfc_ref, gcc_ref, sf_ref, sc_ref,
               aff, afc, acc_, asf, asc):
    b = pl.program_id(0)
    c = c_ref[0]
    p = p_ref[0]
    t = jax.lax.dot_general(c, p, (((1,), (0,)), ((), ())),
                            preferred_element_type=jnp.float32)
    d = -2.0 * t
    src2 = jnp.sum(c * c, axis=1, keepdims=True)
    d = d + src2
    dst2 = jnp.sum(p * p, axis=0, keepdims=True)
    d = d + dst2
    maskv = jnp.where(d <= RADIUS * RADIUS, 1.0, 0.0).astype(jnp.float32)

    oh = (jax.lax.broadcasted_iota(jnp.int32, (NPOINT, N), 1)
          == f_ref[0]).astype(jnp.float32)
    lo = jax.lax.dot_general(slo_ref[0], oh, (((1,), (1,)), ((), ())),
                             preferred_element_type=jnp.float32)
    hi = jax.lax.dot_general(shi_ref[0], oh, (((1,), (1,)), ((), ())),
                             preferred_element_type=jnp.float32)
    sg_ref[0] = hi.astype(jnp.int32) * 256 + lo.astype(jnp.int32)

    lane = jax.lax.broadcasted_iota(jnp.int32, (NPOINT, N), 1)
    lmod = lane & 15
    cw = maskv
    for k in (1, 2, 4, 8):
        sh = jnp.concatenate(
            [jnp.zeros((NPOINT, k), jnp.float32), cw[:, :N - k]], axis=1)
        cw = cw + jnp.where(lmod >= k, sh, 0.0)
    cs_m = (jax.lax.broadcasted_iota(jnp.int32, (N, NCH), 0) // 16
            == jax.lax.broadcasted_iota(jnp.int32, (N, NCH), 1)
            ).astype(jnp.float32)
    c128 = jax.lax.dot_general(maskv, cs_m, (((1,), (0,)), ((), ())),
                               preferred_element_type=jnp.float32)
    lts = (jax.lax.broadcasted_iota(jnp.int32, (NCH, NCH), 0)
           < jax.lax.broadcasted_iota(jnp.int32, (NCH, NCH), 1)
           ).astype(jnp.float32)
    coff = jax.lax.dot_general(c128, lts, (((1,), (0,)), ((), ())),
                               preferred_element_type=jnp.float32)
    chi = jnp.floor(coff * (1.0 / 256.0))
    clo = coff - chi * 256.0
    er = (jax.lax.broadcasted_iota(jnp.int32, (NCH, N), 0)
          == jax.lax.broadcasted_iota(jnp.int32, (NCH, N), 1) // 16
          ).astype(jnp.float32)
    exp = (jax.lax.dot_general(chi, er, (((1,), (0,)), ((), ())),
                               preferred_element_type=jnp.float32) * 256.0
           + jax.lax.dot_general(clo, er, (((1,), (0,)), ((), ())),
                                 preferred_element_type=jnp.float32))
    cincl = cw + exp

    out32 = jax.lax.broadcasted_iota(jnp.int32, (NPOINT, NSAMPLE), 1)

    def body(k, r):
        kf = jnp.float32(1.0) * k
        rk = jnp.sum(jnp.where(cincl <= kf, 1.0, 0.0), axis=1, keepdims=True)
        return jnp.where(out32 == k, rk, r)

    r0 = maskv[:, :NSAMPLE] * 0.0
    r = jax.lax.fori_loop(0, NSAMPLE, body, r0)
    first = jnp.sum(jnp.where(out32 == 0, r, 0.0), axis=1, keepdims=True)
    idxf = jnp.where(r == jnp.float32(N), first, r)
    idxf = idxf + jnp.float32(1.0) * (b * N)
    idx_ref[0] = idxf.astype(jnp.int32)

    ft = ft_ref[0]
    cmat = c
    selmat = maskv * jnp.where(cincl <= jnp.float32(NSAMPLE), 1.0, 0.0)
    tot = cincl[:, N - 1:N]
    padcnt = jnp.maximum(jnp.float32(NSAMPLE) - tot, 0.0)
    firstoh = maskv * jnp.where(cincl == 1.0, 1.0, 0.0)
    wmat = selmat + firstoh * padcnt
    onesc = jnp.zeros((NPOINT, 1), jnp.float32) + 1.0
    multc = jax.lax.dot_general(wmat, onesc, (((0,), (0,)), ((), ())),
                                preferred_element_type=jnp.float32)
    gffc = jax.lax.dot_general(ft * multc, ft, (((0,), (0,)), ((), ())),
                               preferred_element_type=jnp.float32)
    fs = jax.lax.dot_general(wmat, ft, (((1,), (0,)), ((), ())),
                             preferred_element_type=jnp.float32)
    gfcc = jax.lax.dot_general(fs, cmat, (((0,), (0,)), ((), ())),
                               preferred_element_type=jnp.float32)
    gccc = jax.lax.dot_general(cmat, cmat, (((0,), (0,)), ((), ())),
                               preferred_element_type=jnp.float32) * \
        jnp.float32(NSAMPLE)
    sfc = jnp.sum(fs, axis=0, keepdims=True)
    scc = jnp.sum(cmat, axis=0, keepdims=True) * jnp.float32(NSAMPLE)

    @pl.when(b == 0)
    def _():
        aff[...] = jnp.zeros_like(aff)
        afc[...] = jnp.zeros_like(afc)
        acc_[...] = jnp.zeros_like(acc_)
        asf[...] = jnp.zeros_like(asf)
        asc[...] = jnp.zeros_like(asc)

    aff[...] += gffc
    afc[...] += gfcc
    acc_[...] += gccc
    asf[...] += sfc
    asc[...] += scc

    @pl.when(b == B - 1)
    def _():
        gff_ref[...] = aff[...]
        gfc_ref[...] = afc[...]
        gcc_ref[...] = acc_[...]
        sf_ref[...] = asf[...]
        sc_ref[...] = asc[...]


def _run_mask(new_xyz_t, xyz, fps3, seeds_lo, seeds_hi, feat3):
    return pl.pallas_call(
        _mask_body,
        grid=(B,),
        in_specs=[
            pl.BlockSpec((1, NPOINT, 3), lambda b: (b, 0, 0)),
            pl.BlockSpec((1, 3, N), lambda b: (b, 0, 0)),
            pl.BlockSpec((1, NPOINT, 1), lambda b: (b, 0, 0)),
            pl.BlockSpec((1, 1, N), lambda b: (b, 0, 0)),
            pl.BlockSpec((1, 1, N), lambda b: (b, 0, 0)),
            pl.BlockSpec((1, N, FW), lambda b: (b, 0, 0)),
        ],
        out_specs=[
            pl.BlockSpec((1, 1, NPOINT), lambda b: (b, 0, 0)),
            pl.BlockSpec((1, NPOINT, NSAMPLE), lambda b: (b, 0, 0)),
            pl.BlockSpec((FW, FW), lambda b: (0, 0)),
            pl.BlockSpec((FW, 3), lambda b: (0, 0)),
            pl.BlockSpec((3, 3), lambda b: (0, 0)),
            pl.BlockSpec((1, FW), lambda b: (0, 0)),
            pl.BlockSpec((1, 3), lambda b: (0, 0)),
        ],
        out_shape=[
            jax.ShapeDtypeStruct((B, 1, NPOINT), jnp.int32),
            jax.ShapeDtypeStruct((B, NPOINT, NSAMPLE), jnp.int32),
            jax.ShapeDtypeStruct((FW, FW), jnp.float32),
            jax.ShapeDtypeStruct((FW, 3), jnp.float32),
            jax.ShapeDtypeStruct((3, 3), jnp.float32),
            jax.ShapeDtypeStruct((1, FW), jnp.float32),
            jax.ShapeDtypeStruct((1, 3), jnp.float32),
        ],
        scratch_shapes=[
            pltpu.VMEM((FW, FW), jnp.float32),
            pltpu.VMEM((FW, 3), jnp.float32),
            pltpu.VMEM((3, 3), jnp.float32),
            pltpu.VMEM((1, FW), jnp.float32),
            pltpu.VMEM((1, 3), jnp.float32),
        ],
    )(new_xyz_t, xyz, fps3, seeds_lo, seeds_hi, feat3)


GROUP = 128
NGRP = RPW * NSAMPLE // GROUP


def _sc_body(idx_hbm, feat_hbm, x_hbm, ib0, ib1, rb0, rb1, sem0, sem1):
    cid = lax.axis_index("c")
    sid = lax.axis_index("s")
    w = sid * 2 + cid
    base = w * RPW * NSAMPLE

    pltpu.sync_copy(idx_hbm.at[pl.ds(base, GROUP)], ib0)
    pltpu.async_copy(feat_hbm.at[ib0], rb0, sem0)

    def g_it(h, carry):
        g0 = 2 * h
        pltpu.sync_copy(idx_hbm.at[pl.ds(base + (g0 + 1) * GROUP, GROUP)], ib1)
        pltpu.async_copy(feat_hbm.at[ib1], rb1, sem1)
        pltpu.make_async_copy(feat_hbm.at[ib0], rb0, sem0).wait()
        pltpu.sync_copy(rb0, x_hbm.at[pl.ds(base + g0 * GROUP, GROUP)])

        @pl.when(g0 + 2 < NGRP)
        def _():
            pltpu.sync_copy(idx_hbm.at[pl.ds(base + (g0 + 2) * GROUP, GROUP)],
                            ib0)
            pltpu.async_copy(feat_hbm.at[ib0], rb0, sem0)

        pltpu.make_async_copy(feat_hbm.at[ib1], rb1, sem1).wait()
        pltpu.sync_copy(rb1, x_hbm.at[pl.ds(base + (g0 + 1) * GROUP, GROUP)])
        return carry

    lax.fori_loop(0, NGRP // 2, g_it, 0)


def _run_sc(idxflat, feat):
    mesh = plsc.VectorSubcoreMesh(core_axis_name="c", subcore_axis_name="s")
    f = pl.kernel(
        _sc_body,
        out_type=jax.ShapeDtypeStruct((P, FW), jnp.float32),
        mesh=mesh,
        scratch_types=[
            pltpu.VMEM((GROUP,), jnp.int32),
            pltpu.VMEM((GROUP,), jnp.int32),
            pltpu.VMEM((GROUP, FW), jnp.float32),
            pltpu.VMEM((GROUP, FW), jnp.float32),
            pltpu.SemaphoreType.DMA,
            pltpu.SemaphoreType.DMA,
        ],
    )
    return f(idxflat, feat)


def _expand_mat():
    return (jax.lax.broadcasted_iota(jnp.int32, (RB, GB), 0) // NSAMPLE
            == jax.lax.broadcasted_iota(jnp.int32, (RB, GB), 1)
            ).astype(jnp.float32)


def _m1_body(x_ref, c_ref, gff_ref, gfc_ref, gcc_ref, sf_ref, sc_ref,
             aff, afc, acc_, asf, asc):
    g = pl.program_id(0)
    x = x_ref[...]
    ce = jax.lax.dot_general(_expand_mat(), c_ref[...], (((1,), (0,)), ((), ())),
                             preferred_element_type=jnp.float32)
    gffc = jax.lax.dot_general(x, x, (((0,), (0,)), ((), ())),
                               preferred_element_type=jnp.float32)
    gfcc = jax.lax.dot_general(x, ce, (((0,), (0,)), ((), ())),
                               preferred_element_type=jnp.float32)
    gccc = jax.lax.dot_general(ce, ce, (((0,), (0,)), ((), ())),
                               preferred_element_type=jnp.float32)
    sfc = jnp.sum(x, axis=0, keepdims=True)
    scc = jnp.sum(ce, axis=0, keepdims=True)

    @pl.when(g == 0)
    def _():
        aff[...] = jnp.zeros_like(aff)
        afc[...] = jnp.zeros_like(afc)
        acc_[...] = jnp.zeros_like(acc_)
        asf[...] = jnp.zeros_like(asf)
        asc[...] = jnp.zeros_like(asc)

    aff[...] += gffc
    afc[...] += gfcc
    acc_[...] += gccc
    asf[...] += sfc
    asc[...] += scc

    @pl.when(g == NBLK - 1)
    def _():
        gff_ref[...] = aff[...]
        gfc_ref[...] = afc[...]
        gcc_ref[...] = acc_[...]
        sf_ref[...] = asf[...]
        sc_ref[...] = asc[...]


def _run_m1(X, cxyz):
    return pl.pallas_call(
        _m1_body,
        grid=(NBLK,),
        in_specs=[
            pl.BlockSpec((RB, FW), lambda g: (g, 0)),
            pl.BlockSpec((GB, 3), lambda g: (g, 0)),
        ],
        out_specs=[
            pl.BlockSpec((FW, FW), lambda g: (0, 0)),
            pl.BlockSpec((FW, 3), lambda g: (0, 0)),
            pl.BlockSpec((3, 3), lambda g: (0, 0)),
            pl.BlockSpec((1, FW), lambda g: (0, 0)),
            pl.BlockSpec((1, 3), lambda g: (0, 0)),
        ],
        out_shape=[
            jax.ShapeDtypeStruct((FW, FW), jnp.float32),
            jax.ShapeDtypeStruct((FW, 3), jnp.float32),
            jax.ShapeDtypeStruct((3, 3), jnp.float32),
            jax.ShapeDtypeStruct((1, FW), jnp.float32),
            jax.ShapeDtypeStruct((1, 3), jnp.float32),
        ],
        scratch_shapes=[
            pltpu.VMEM((FW, FW), jnp.float32),
            pltpu.VMEM((FW, 3), jnp.float32),
            pltpu.VMEM((3, 3), jnp.float32),
            pltpu.VMEM((1, FW), jnp.float32),
            pltpu.VMEM((1, 3), jnp.float32),
        ],
    )(X, cxyz)


def _mlp0_body(x_ref, c_ref, vf_ref, vc_ref, b_ref, gm_ref, be_ref,
               gff_ref, gfc_ref, gcc_ref, sf_ref, sc_ref,
               z_ref, g1_ref, s1_ref, wf_s, wc_s, brow_s, ag1, as1):
    g = pl.program_id(0)
    cout = vf_ref.shape[0]

    @pl.when(g == 0)
    def _():
        vf = vf_ref[...]
        vc = vc_ref[...]
        invp = jnp.float32(1.0 / P)
        ml = (lax.dot_general(sf_ref[...], vf, (((1,), (1,)), ((), ())),
                              preferred_element_type=jnp.float32)
              + lax.dot_general(sc_ref[...], vc, (((1,), (1,)), ((), ())),
                                preferred_element_type=jnp.float32)) * invp
        t1 = lax.dot_general(vf, gff_ref[...], (((1,), (0,)), ((), ())),
                             preferred_element_type=jnp.float32)
        q = jnp.sum(t1 * vf, axis=1, keepdims=True)
        t2 = lax.dot_general(vf, gfc_ref[...], (((1,), (0,)), ((), ())),
                             preferred_element_type=jnp.float32)
        q = q + 2.0 * jnp.sum(t2 * vc, axis=1, keepdims=True)
        t3 = lax.dot_general(vc, gcc_ref[...], (((1,), (0,)), ((), ())),
                             preferred_element_type=jnp.float32)
        q = q + jnp.sum(t3 * vc, axis=1, keepdims=True)
        eye = _eye(cout)
        qrow = lax.dot_general(q, eye, (((0,), (0,)), ((), ())),
                               preferred_element_type=jnp.float32)
        var = qrow * invp - ml * ml
        a = gm_ref[...] / jnp.sqrt(var + 1e-5)
        brow_s[...] = a * (-ml) + be_ref[...]
        acol = lax.dot_general(eye, a, (((1,), (1,)), ((), ())),
                               preferred_element_type=jnp.float32)
        wf_s[...] = vf_ref[...] * acol
        wc_s[...] = vc_ref[...] * acol

    x = x_ref[...]
    ce = jax.lax.dot_general(_expand_mat(), c_ref[...], (((1,), (0,)), ((), ())),
                             preferred_element_type=jnp.float32)
    y = (lax.dot_general(x, wf_s[...], (((1,), (1,)), ((), ())),
                         preferred_element_type=jnp.float32)
         + lax.dot_general(ce, wc_s[...], (((1,), (1,)), ((), ())),
                           preferred_element_type=jnp.float32)
         + brow_s[...])
    z = jnp.maximum(y, 0.0)
    z_ref[...] = z
    g1c = lax.dot_general(z, z, (((0,), (0,)), ((), ())),
                          preferred_element_type=jnp.float32)
    s1c = jnp.sum(z, axis=0, keepdims=True)

    @pl.when(g == 0)
    def _():
        ag1[...] = jnp.zeros_like(ag1)
        as1[...] = jnp.zeros_like(as1)

    ag1[...] += g1c
    as1[...] += s1c

    @pl.when(g == NBLK - 1)
    def _():
        g1_ref[...] = ag1[...]
        s1_ref[...] = as1[...]


def _run_mlp0(x, cxyz, vf, vc, brow, grow, berow, gff, gfc, gcc, sf, sc_):
    cout = vf.shape[0]
    return pl.pallas_call(
        _mlp0_body,
        grid=(NBLK,),
        in_specs=[
            pl.BlockSpec((RB, FW), lambda g: (g, 0)),
            pl.BlockSpec((GB, 3), lambda g: (g, 0)),
            pl.BlockSpec((cout, FW), lambda g: (0, 0)),
            pl.BlockSpec((cout, 3), lambda g: (0, 0)),
            pl.BlockSpec((1, cout), lambda g: (0, 0)),
            pl.BlockSpec((1, cout), lambda g: (0, 0)),
            pl.BlockSpec((1, cout), lambda g: (0, 0)),
            pl.BlockSpec((FW, FW), lambda g: (0, 0)),
            pl.BlockSpec((FW, 3), lambda g: (0, 0)),
            pl.BlockSpec((3, 3), lambda g: (0, 0)),
            pl.BlockSpec((1, FW), lambda g: (0, 0)),
            pl.BlockSpec((1, 3), lambda g: (0, 0)),
        ],
        out_specs=[
            pl.BlockSpec((RB, cout), lambda g: (g, 0)),
            pl.BlockSpec((cout, cout), lambda g: (0, 0)),
            pl.BlockSpec((1, cout), lambda g: (0, 0)),
        ],
        out_shape=[
            jax.ShapeDtypeStruct((P, cout), jnp.float32),
            jax.ShapeDtypeStruct((cout, cout), jnp.float32),
            jax.ShapeDtypeStruct((1, cout), jnp.float32),
        ],
        scratch_shapes=[
            pltpu.VMEM((cout, FW), jnp.float32),
            pltpu.VMEM((cout, 3), jnp.float32),
            pltpu.VMEM((1, cout), jnp.float32),
            pltpu.VMEM((cout, cout), jnp.float32),
            pltpu.VMEM((1, cout), jnp.float32),
        ],
    )(x, cxyz, vf, vc, brow, grow, berow, gff, gfc, gcc, sf, sc_)


def _eye(n):
    return (jax.lax.broadcasted_iota(jnp.int32, (n, n), 0)
            == jax.lax.broadcasted_iota(jnp.int32, (n, n), 1)).astype(jnp.float32)


def _mlp_body(cin, cout, last, x_ref, v_ref, b_ref, gm_ref, be_ref,
              gp_ref, sp_ref, *rest):
    if last:
        z_ref, wf_s, brow_s = rest
    else:
        z_ref, g1_ref, s1_ref, wf_s, brow_s, ag1, as1 = rest
    g = pl.program_id(0)

    @pl.when(g == 0)
    def _():
        v = v_ref[...]
        gp = gp_ref[...]
        sp = sp_ref[...]
        invp = jnp.float32(1.0 / P)
        ml = lax.dot_general(sp, v, (((1,), (1,)), ((), ())),
                             preferred_element_type=jnp.float32) * invp
        t1 = lax.dot_general(v, gp, (((1,), (0,)), ((), ())),
                             preferred_element_type=jnp.float32)
        q = jnp.sum(t1 * v, axis=1, keepdims=True)
        eye = _eye(cout)
        qrow = lax.dot_general(q, eye, (((0,), (0,)), ((), ())),
                               preferred_element_type=jnp.float32)
        var = qrow * invp - ml * ml
        a = gm_ref[...] / jnp.sqrt(var + 1e-5)
        mean = ml + b_ref[...]
        brow_s[...] = a * (b_ref[...] - mean) + be_ref[...]
        acol = lax.dot_general(eye, a, (((1,), (1,)), ((), ())),
                               preferred_element_type=jnp.float32)
        wf_s[...] = v * acol

    x = x_ref[...]
    y = lax.dot_general(x, wf_s[...], (((1,), (1,)), ((), ())),
                        preferred_element_type=jnp.float32) + brow_s[...]
    z = jnp.maximum(y, 0.0)

    if last:
        z_ref[...] = jnp.max(z.reshape(GB, NSAMPLE, cout), axis=1)
    else:
        z_ref[...] = z
        g1c = lax.dot_general(z, z, (((0,), (0,)), ((), ())),
                              preferred_element_type=jnp.float32)
        s1c = jnp.sum(z, axis=0, keepdims=True)

        @pl.when(g == 0)
        def _():
            ag1[...] = jnp.zeros_like(ag1)
            as1[...] = jnp.zeros_like(as1)

        ag1[...] += g1c
        as1[...] += s1c

        @pl.when(g == NBLK - 1)
        def _():
            g1_ref[...] = ag1[...]
            s1_ref[...] = as1[...]


def _run_mlp_layer(x, v, brow, grow, berow, gp, sp, last=False):
    cin = x.shape[1]
    cout = v.shape[0]
    small = [
        pl.BlockSpec((cout, cin), lambda g: (0, 0)),
        pl.BlockSpec((1, cout), lambda g: (0, 0)),
        pl.BlockSpec((1, cout), lambda g: (0, 0)),
        pl.BlockSpec((1, cout), lambda g: (0, 0)),
        pl.BlockSpec((cin, cin), lambda g: (0, 0)),
        pl.BlockSpec((1, cin), lambda g: (0, 0)),
    ]
    if last:
        out_specs = pl.BlockSpec((GB, cout), lambda g: (g, 0))
        out_shape = jax.ShapeDtypeStruct((NROWS, cout), jnp.float32)
        scratch = [pltpu.VMEM((cout, cin), jnp.float32),
                   pltpu.VMEM((1, cout), jnp.float32)]
    else:
        out_specs = [
            pl.BlockSpec((RB, cout), lambda g: (g, 0)),
            pl.BlockSpec((cout, cout), lambda g: (0, 0)),
            pl.BlockSpec((1, cout), lambda g: (0, 0)),
        ]
        out_shape = [
            jax.ShapeDtypeStruct((P, cout), jnp.float32),
            jax.ShapeDtypeStruct((cout, cout), jnp.float32),
            jax.ShapeDtypeStruct((1, cout), jnp.float32),
        ]
        scratch = [pltpu.VMEM((cout, cin), jnp.float32),
                   pltpu.VMEM((1, cout), jnp.float32),
                   pltpu.VMEM((cout, cout), jnp.float32),
                   pltpu.VMEM((1, cout), jnp.float32)]

    def body(*refs):
        _mlp_body(cin, cout, last, *refs)

    return pl.pallas_call(
        body,
        grid=(NBLK,),
        in_specs=[pl.BlockSpec((RB, cin), lambda g: (g, 0))] + small,
        out_specs=out_specs,
        out_shape=out_shape,
        scratch_shapes=scratch,
    )(x, v, brow, grow, berow, gp, sp)


def kernel(xyz, points, seed_inds, W0, b0, gamma0, beta0,
           W1, b1, gamma1, beta1, W2, b2, gamma2, beta2):
    fps_idx, nx, ny, nz = _run_fps(xyz)
    new_xyz_t = jnp.stack([nx, ny, nz], axis=-1)
    seeds32 = seed_inds.astype(jnp.int32)
    seeds_lo = (seeds32 % 256).astype(jnp.float32).reshape(B, 1, N)
    seeds_hi = (seeds32 // 256).astype(jnp.float32).reshape(B, 1, N)
    feat3 = jnp.concatenate(
        [xyz.transpose(0, 2, 1), points.transpose(0, 2, 1),
         jnp.zeros((B, N, FW - 67), jnp.float32)], axis=-1)
    seed_g, idx_out, Gff, Gfc, Gcc, sf, sc_ = _run_mask(
        new_xyz_t, xyz, fps_idx.reshape(B, NPOINT, 1),
        seeds_lo, seeds_hi, feat3)

    X = _run_sc(idx_out.reshape(-1), feat3.reshape(B * N, FW))
    cxyz = new_xyz_t.reshape(NROWS, 3)

    co0 = W0.shape[0]
    Vf = jnp.concatenate(
        [W0, jnp.zeros((co0, FW - 67), jnp.float32)], axis=1)
    Vc = -W0[:, :3]
    z0, G1, s1 = _run_mlp0(X, cxyz, Vf, Vc, b0.reshape(1, -1),
                           gamma0.reshape(1, -1), beta0.reshape(1, -1),
                           Gff, Gfc, Gcc, sf, sc_)
    z1, G2, s2 = _run_mlp_layer(z0, W1, b1.reshape(1, -1),
                                gamma1.reshape(1, -1), beta1.reshape(1, -1),
                                G1, s1)
    Y = _run_mlp_layer(z1, W2, b2.reshape(1, -1),
                       gamma2.reshape(1, -1), beta2.reshape(1, -1),
                       G2, s2, last=True)

    new_points_out = Y.reshape(B, NPOINT, W2.shape[0]).transpose(0, 2, 1)
    new_xyz_out = jnp.stack([nx, ny, nz], axis=1)
    this_seed = seed_g.reshape(B, NPOINT).astype(seed_inds.dtype)
    return (new_xyz_out, new_points_out, this_seed)


# speedup vs baseline: 1.0497x; 1.0497x over previous
import jax
import jax.numpy as jnp
from jax import lax
from jax.experimental import pallas as pl
from jax.experimental.pallas import tpu as pltpu
from jax.experimental.pallas import tpu_sc as plsc

B, N, DPTS = 8, 2048, 64
NPOINT, RADIUS, NSAMPLE = 512, 0.2, 32
FW = 128
NROWS = B * NPOINT
P = NROWS * NSAMPLE
NW = 32
RPW = NROWS // NW
RB = 4096
GB = RB // NSAMPLE
NBLK = P // RB


NGF = 2
HB = B // NGF


def _fps_body(a_ref, idx_ref, cx_ref, cy_ref, cz_ref):
    a = a_ref[...]
    lane = jax.lax.broadcasted_iota(jnp.int32, (HB, N), 1)
    lane3 = jax.lax.broadcasted_iota(jnp.int32, (3 * HB, N), 1)
    out_lane = jax.lax.broadcasted_iota(jnp.int32, (HB, NPOINT), 1)
    slabs = [a[3 * HB * g:3 * HB * (g + 1)] for g in range(NGF)]

    def step_grp(slab, st, i):
        dist, far, cents, ox, oy, oz = st
        far3 = jnp.concatenate([far, far, far], axis=0)
        oh = lane3 == far3
        cvec = jnp.sum(jnp.where(oh, slab, 0.0), axis=1, keepdims=True)
        sel = out_lane == i
        cents = jnp.where(sel, far, cents)
        ox = jnp.where(sel, cvec[0:HB], ox)
        oy = jnp.where(sel, cvec[HB:2 * HB], oy)
        oz = jnp.where(sel, cvec[2 * HB:3 * HB], oz)
        df = slab - cvec
        sq = df * df
        d = sq[0:HB] + sq[HB:2 * HB]
        d = d + sq[2 * HB:3 * HB]
        dist = jnp.minimum(dist, d)
        m = jnp.max(dist, axis=1, keepdims=True)
        far = jnp.min(jnp.where(dist == m, lane, N), axis=1, keepdims=True)
        return (dist, far, cents, ox, oy, oz)

    def body(i, sts):
        return tuple(step_grp(slabs[g], sts[g], i) for g in range(NGF))

    def init(slab):
        dist0 = jnp.full((HB, N), 1e10, dtype=jnp.float32) + slab[0:HB] * 0.0
        far0 = jnp.min(lane, axis=1, keepdims=True)
        ox0 = slab[0:HB, :NPOINT] * 0.0
        return (dist0, far0, ox0.astype(jnp.int32), ox0, ox0, ox0)

    sts = jax.lax.fori_loop(0, NPOINT, body,
                            tuple(init(slabs[g]) for g in range(NGF)))
    idx_ref[...] = jnp.concatenate([st[2] for st in sts], axis=0)
    cx_ref[...] = jnp.concatenate([st[3] for st in sts], axis=0)
    cy_ref[...] = jnp.concatenate([st[4] for st in sts], axis=0)
    cz_ref[...] = jnp.concatenate([st[5] for st in sts], axis=0)


def _run_fps(xyz):
    a = xyz.reshape(NGF, HB, 3, N).transpose(0, 2, 1, 3).reshape(3 * B, N)
    out_shape = [
        jax.ShapeDtypeStruct((B, NPOINT), jnp.int32),
        jax.ShapeDtypeStruct((B, NPOINT), jnp.float32),
        jax.ShapeDtypeStruct((B, NPOINT), jnp.float32),
        jax.ShapeDtypeStruct((B, NPOINT), jnp.float32),
    ]
    return pl.pallas_call(_fps_body, out_shape=out_shape)(a)


NCH = N // 16


def _mask_body(c_ref, p_ref, f_ref, slo_ref, shi_ref, ft_ref,
               sg_ref, idx_ref, gff_ref, gfc_ref, gcc_ref, sf_ref, sc_ref,
               aff, afc, acc_, asf, asc):
    b = pl.program_id(0)
    c = c_ref[0]
    p = p_ref[0]
    t = jax.lax.dot_general(c, p, (((1,), (0,)), ((), ())),
                            preferred_element_type=jnp.float32)
    d = -2.0 * t
    src2 = jnp.sum(c * c, axis=1, keepdims=True)
    d = d + src2
    dst2 = jnp.sum(p * p, axis=0, keepdims=True)
    d = d + dst2
    maskv = jnp.where(d <= RADIUS * RADIUS, 1.0, 0.0).astype(jnp.float32)

    oh = (jax.lax.broadcasted_iota(jnp.int32, (NPOINT, N), 1)
          == f_ref[0]).astype(jnp.float32)
    lo = jax.lax.dot_general(slo_ref[0], oh, (((1,), (1,)), ((), ())),
                             preferred_element_type=jnp.float32)
    hi = jax.lax.dot_general(shi_ref[0], oh, (((1,), (1,)), ((), ())),
                             preferred_element_type=jnp.float32)
    sg_ref[0] = hi.astype(jnp.int32) * 256 + lo.astype(jnp.int32)

    lane = jax.lax.broadcasted_iota(jnp.int32, (NPOINT, N), 1)
    lmod = lane & 15
    cw = maskv
    for k in (1, 2, 4, 8):
        sh = jnp.concatenate(
            [jnp.zeros((NPOINT, k), jnp.float32), cw[:, :N - k]], axis=1)
        cw = cw + jnp.where(lmod >= k, sh, 0.0)
    cs_m = (jax.lax.broadcasted_iota(jnp.int32, (N, NCH), 0) // 16
            == jax.lax.broadcasted_iota(jnp.int32, (N, NCH), 1)
            ).astype(jnp.float32)
    c128 = jax.lax.dot_general(maskv, cs_m, (((1,), (0,)), ((), ())),
                               preferred_element_type=jnp.float32)
    lts = (jax.lax.broadcasted_iota(jnp.int32, (NCH, NCH), 0)
           < jax.lax.broadcasted_iota(jnp.int32, (NCH, NCH), 1)
           ).astype(jnp.float32)
    coff = jax.lax.dot_general(c128, lts, (((1,), (0,)), ((), ())),
                               preferred_element_type=jnp.float32)
    chi = jnp.floor(coff * (1.0 / 256.0))
    clo = coff - chi * 256.0
    er = (jax.lax.broadcasted_iota(jnp.int32, (NCH, N), 0)
          == jax.lax.broadcasted_iota(jnp.int32, (NCH, N), 1) // 16
          ).astype(jnp.float32)
    exp = (jax.lax.dot_general(chi, er, (((1,), (0,)), ((), ())),
                               preferred_element_type=jnp.float32) * 256.0
           + jax.lax.dot_general(clo, er, (((1,), (0,)), ((), ())),
                                 preferred_element_type=jnp.float32))
    cincl = cw + exp

    out32 = jax.lax.broadcasted_iota(jnp.int32, (NPOINT, NSAMPLE), 1)

    def body(k, r):
        kf = jnp.float32(1.0) * k
        rk = jnp.sum(jnp.where(cincl <= kf, 1.0, 0.0), axis=1, keepdims=True)
        return jnp.where(out32 == k, rk, r)

    r0 = maskv[:, :NSAMPLE] * 0.0
    r = jax.lax.fori_loop(0, NSAMPLE, body, r0)
    first = jnp.sum(jnp.where(out32 == 0, r, 0.0), axis=1, keepdims=True)
    idxf = jnp.where(r == jnp.float32(N), first, r)
    idxf = idxf + jnp.float32(1.0) * (b * N)
    idx_ref[0] = idxf.astype(jnp.int32)

    ft = ft_ref[0]
    cmat = c
    selmat = maskv * jnp.where(cincl <= jnp.float32(NSAMPLE), 1.0, 0.0)
    tot = cincl[:, N - 1:N]
    padcnt = jnp.maximum(jnp.float32(NSAMPLE) - tot, 0.0)
    firstoh = maskv * jnp.where(cincl == 1.0, 1.0, 0.0)
    wmat = selmat + firstoh * padcnt
    onesc = jnp.zeros((NPOINT, 1), jnp.float32) + 1.0
    multc = jax.lax.dot_general(wmat, onesc, (((0,), (0,)), ((), ())),
                                preferred_element_type=jnp.float32)
    gffc = jax.lax.dot_general(ft * multc, ft, (((0,), (0,)), ((), ())),
                               preferred_element_type=jnp.float32)
    fs = jax.lax.dot_general(wmat, ft, (((1,), (0,)), ((), ())),
                             preferred_element_type=jnp.float32)
    gfcc = jax.lax.dot_general(fs, cmat, (((0,), (0,)), ((), ())),
                               preferred_element_type=jnp.float32)
    gccc = jax.lax.dot_general(cmat, cmat, (((0,), (0,)), ((), ())),
                               preferred_element_type=jnp.float32) * \
        jnp.float32(NSAMPLE)
    sfc = jnp.sum(fs, axis=0, keepdims=True)
    scc = jnp.sum(cmat, axis=0, keepdims=True) * jnp.float32(NSAMPLE)

    @pl.when(b == 0)
    def _():
        aff[...] = jnp.zeros_like(aff)
        afc[...] = jnp.zeros_like(afc)
        acc_[...] = jnp.zeros_like(acc_)
        asf[...] = jnp.zeros_like(asf)
        asc[...] = jnp.zeros_like(asc)

    aff[...] += gffc
    afc[...] += gfcc
    acc_[...] += gccc
    asf[...] += sfc
    asc[...] += scc

    @pl.when(b == B - 1)
    def _():
        gff_ref[...] = aff[...]
        gfc_ref[...] = afc[...]
        gcc_ref[...] = acc_[...]
        sf_ref[...] = asf[...]
        sc_ref[...] = asc[...]


def _run_mask(new_xyz_t, xyz, fps3, seeds_lo, seeds_hi, feat3):
    return pl.pallas_call(
        _mask_body,
        grid=(B,),
        in_specs=[
            pl.BlockSpec((1, NPOINT, 3), lambda b: (b, 0, 0)),
            pl.BlockSpec((1, 3, N), lambda b: (b, 0, 0)),
            pl.BlockSpec((1, NPOINT, 1), lambda b: (b, 0, 0)),
            pl.BlockSpec((1, 1, N), lambda b: (b, 0, 0)),
            pl.BlockSpec((1, 1, N), lambda b: (b, 0, 0)),
            pl.BlockSpec((1, N, FW), lambda b: (b, 0, 0)),
        ],
        out_specs=[
            pl.BlockSpec((1, 1, NPOINT), lambda b: (b, 0, 0)),
            pl.BlockSpec((1, NPOINT, NSAMPLE), lambda b: (b, 0, 0)),
            pl.BlockSpec((FW, FW), lambda b: (0, 0)),
            pl.BlockSpec((FW, 3), lambda b: (0, 0)),
            pl.BlockSpec((3, 3), lambda b: (0, 0)),
            pl.BlockSpec((1, FW), lambda b: (0, 0)),
            pl.BlockSpec((1, 3), lambda b: (0, 0)),
        ],
        out_shape=[
            jax.ShapeDtypeStruct((B, 1, NPOINT), jnp.int32),
            jax.ShapeDtypeStruct((B, NPOINT, NSAMPLE), jnp.int32),
            jax.ShapeDtypeStruct((FW, FW), jnp.float32),
            jax.ShapeDtypeStruct((FW, 3), jnp.float32),
            jax.ShapeDtypeStruct((3, 3), jnp.float32),
            jax.ShapeDtypeStruct((1, FW), jnp.float32),
            jax.ShapeDtypeStruct((1, 3), jnp.float32),
        ],
        scratch_shapes=[
            pltpu.VMEM((FW, FW), jnp.float32),
            pltpu.VMEM((FW, 3), jnp.float32),
            pltpu.VMEM((3, 3), jnp.float32),
            pltpu.VMEM((1, FW), jnp.float32),
            pltpu.VMEM((1, 3), jnp.float32),
        ],
    )(new_xyz_t, xyz, fps3, seeds_lo, seeds_hi, feat3)


GROUP = 128
NGRP = RPW * NSAMPLE // GROUP


def _sc_body(idx_hbm, feat_hbm, x_hbm, ib0, ib1, rb0, rb1, sem0, sem1):
    cid = lax.axis_index("c")
    sid = lax.axis_index("s")
    w = sid * 2 + cid
    base = w * RPW * NSAMPLE

    pltpu.sync_copy(idx_hbm.at[pl.ds(base, GROUP)], ib0)
    pltpu.async_copy(feat_hbm.at[ib0], rb0, sem0)

    def g_it(h, carry):
        g0 = 2 * h
        pltpu.sync_copy(idx_hbm.at[pl.ds(base + (g0 + 1) * GROUP, GROUP)], ib1)
        pltpu.async_copy(feat_hbm.at[ib1], rb1, sem1)
        pltpu.make_async_copy(feat_hbm.at[ib0], rb0, sem0).wait()
        pltpu.sync_copy(rb0, x_hbm.at[pl.ds(base + g0 * GROUP, GROUP)])

        @pl.when(g0 + 2 < NGRP)
        def _():
            pltpu.sync_copy(idx_hbm.at[pl.ds(base + (g0 + 2) * GROUP, GROUP)],
                            ib0)
            pltpu.async_copy(feat_hbm.at[ib0], rb0, sem0)

        pltpu.make_async_copy(feat_hbm.at[ib1], rb1, sem1).wait()
        pltpu.sync_copy(rb1, x_hbm.at[pl.ds(base + (g0 + 1) * GROUP, GROUP)])
        return carry

    lax.fori_loop(0, NGRP // 2, g_it, 0)


def _run_sc(idxflat, feat):
    mesh = plsc.VectorSubcoreMesh(core_axis_name="c", subcore_axis_name="s")
    f = pl.kernel(
        _sc_body,
        out_type=jax.ShapeDtypeStruct((P, FW), jnp.float32),
        mesh=mesh,
        scratch_types=[
            pltpu.VMEM((GROUP,), jnp.int32),
            pltpu.VMEM((GROUP,), jnp.int32),
            pltpu.VMEM((GROUP, FW), jnp.float32),
            pltpu.VMEM((GROUP, FW), jnp.float32),
            pltpu.SemaphoreType.DMA,
            pltpu.SemaphoreType.DMA,
        ],
    )
    return f(idxflat, feat)


def _expand_mat():
    return (jax.lax.broadcasted_iota(jnp.int32, (RB, GB), 0) // NSAMPLE
            == jax.lax.broadcasted_iota(jnp.int32, (RB, GB), 1)
            ).astype(jnp.float32)


def _m1_body(x_ref, c_ref, gff_ref, gfc_ref, gcc_ref, sf_ref, sc_ref,
             aff, afc, acc_, asf, asc):
    g = pl.program_id(0)
    x = x_ref[...]
    ce = jax.lax.dot_general(_expand_mat(), c_ref[...], (((1,), (0,)), ((), ())),
                             preferred_element_type=jnp.float32)
    gffc = jax.lax.dot_general(x, x, (((0,), (0,)), ((), ())),
                               preferred_element_type=jnp.float32)
    gfcc = jax.lax.dot_general(x, ce, (((0,), (0,)), ((), ())),
                               preferred_element_type=jnp.float32)
    gccc = jax.lax.dot_general(ce, ce, (((0,), (0,)), ((), ())),
                               preferred_element_type=jnp.float32)
    sfc = jnp.sum(x, axis=0, keepdims=True)
    scc = jnp.sum(ce, axis=0, keepdims=True)

    @pl.when(g == 0)
    def _():
        aff[...] = jnp.zeros_like(aff)
        afc[...] = jnp.zeros_like(afc)
        acc_[...] = jnp.zeros_like(acc_)
        asf[...] = jnp.zeros_like(asf)
        asc[...] = jnp.zeros_like(asc)

    aff[...] += gffc
    afc[...] += gfcc
    acc_[...] += gccc
    asf[...] += sfc
    asc[...] += scc

    @pl.when(g == NBLK - 1)
    def _():
        gff_ref[...] = aff[...]
        gfc_ref[...] = afc[...]
        gcc_ref[...] = acc_[...]
        sf_ref[...] = asf[...]
        sc_ref[...] = asc[...]


def _run_m1(X, cxyz):
    return pl.pallas_call(
        _m1_body,
        grid=(NBLK,),
        in_specs=[
            pl.BlockSpec((RB, FW), lambda g: (g, 0)),
            pl.BlockSpec((GB, 3), lambda g: (g, 0)),
        ],
        out_specs=[
            pl.BlockSpec((FW, FW), lambda g: (0, 0)),
            pl.BlockSpec((FW, 3), lambda g: (0, 0)),
            pl.BlockSpec((3, 3), lambda g: (0, 0)),
            pl.BlockSpec((1, FW), lambda g: (0, 0)),
            pl.BlockSpec((1, 3), lambda g: (0, 0)),
        ],
        out_shape=[
            jax.ShapeDtypeStruct((FW, FW), jnp.float32),
            jax.ShapeDtypeStruct((FW, 3), jnp.float32),
            jax.ShapeDtypeStruct((3, 3), jnp.float32),
            jax.ShapeDtypeStruct((1, FW), jnp.float32),
            jax.ShapeDtypeStruct((1, 3), jnp.float32),
        ],
        scratch_shapes=[
            pltpu.VMEM((FW, FW), jnp.float32),
            pltpu.VMEM((FW, 3), jnp.float32),
            pltpu.VMEM((3, 3), jnp.float32),
            pltpu.VMEM((1, FW), jnp.float32),
            pltpu.VMEM((1, 3), jnp.float32),
        ],
    )(X, cxyz)


def _mlp0_body(x_ref, c_ref, vf_ref, vc_ref, b_ref, gm_ref, be_ref,
               gff_ref, gfc_ref, gcc_ref, sf_ref, sc_ref,
               z_ref, g1_ref, s1_ref, wf_s, wc_s, brow_s, ag1, as1):
    g = pl.program_id(0)
    cout = vf_ref.shape[0]

    @pl.when(g == 0)
    def _():
        vf = vf_ref[...]
        vc = vc_ref[...]
        invp = jnp.float32(1.0 / P)
        ml = (lax.dot_general(sf_ref[...], vf, (((1,), (1,)), ((), ())),
                              preferred_element_type=jnp.float32)
              + lax.dot_general(sc_ref[...], vc, (((1,), (1,)), ((), ())),
                                preferred_element_type=jnp.float32)) * invp
        t1 = lax.dot_general(vf, gff_ref[...], (((1,), (0,)), ((), ())),
                             preferred_element_type=jnp.float32)
        q = jnp.sum(t1 * vf, axis=1, keepdims=True)
        t2 = lax.dot_general(vf, gfc_ref[...], (((1,), (0,)), ((), ())),
                             preferred_element_type=jnp.float32)
        q = q + 2.0 * jnp.sum(t2 * vc, axis=1, keepdims=True)
        t3 = lax.dot_general(vc, gcc_ref[...], (((1,), (0,)), ((), ())),
                             preferred_element_type=jnp.float32)
        q = q + jnp.sum(t3 * vc, axis=1, keepdims=True)
        eye = _eye(cout)
        qrow = lax.dot_general(q, eye, (((0,), (0,)), ((), ())),
                               preferred_element_type=jnp.float32)
        var = qrow * invp - ml * ml
        a = gm_ref[...] / jnp.sqrt(var + 1e-5)
        brow_s[...] = a * (-ml) + be_ref[...]
        acol = lax.dot_general(eye, a, (((1,), (1,)), ((), ())),
                               preferred_element_type=jnp.float32)
        wf_s[...] = vf_ref[...] * acol
        wc_s[...] = vc_ref[...] * acol

    x = x_ref[...]
    ce = jax.lax.dot_general(_expand_mat(), c_ref[...], (((1,), (0,)), ((), ())),
                             preferred_element_type=jnp.float32)
    y = (lax.dot_general(x, wf_s[...], (((1,), (1,)), ((), ())),
                         preferred_element_type=jnp.float32)
         + lax.dot_general(ce, wc_s[...], (((1,), (1,)), ((), ())),
                           preferred_element_type=jnp.float32)
         + brow_s[...])
    z = jnp.maximum(y, 0.0)
    z_ref[...] = z
    g1c = lax.dot_general(z, z, (((0,), (0,)), ((), ())),
                          preferred_element_type=jnp.float32)
    s1c = jnp.sum(z, axis=0, keepdims=True)

    @pl.when(g == 0)
    def _():
        ag1[...] = jnp.zeros_like(ag1)
        as1[...] = jnp.zeros_like(as1)

    ag1[...] += g1c
    as1[...] += s1c

    @pl.when(g == NBLK - 1)
    def _():
        g1_ref[...] = ag1[...]
        s1_ref[...] = as1[...]


def _run_mlp0(x, cxyz, vf, vc, brow, grow, berow, gff, gfc, gcc, sf, sc_):
    cout = vf.shape[0]
    return pl.pallas_call(
        _mlp0_body,
        grid=(NBLK,),
        in_specs=[
            pl.BlockSpec((RB, FW), lambda g: (g, 0)),
            pl.BlockSpec((GB, 3), lambda g: (g, 0)),
            pl.BlockSpec((cout, FW), lambda g: (0, 0)),
            pl.BlockSpec((cout, 3), lambda g: (0, 0)),
            pl.BlockSpec((1, cout), lambda g: (0, 0)),
            pl.BlockSpec((1, cout), lambda g: (0, 0)),
            pl.BlockSpec((1, cout), lambda g: (0, 0)),
            pl.BlockSpec((FW, FW), lambda g: (0, 0)),
            pl.BlockSpec((FW, 3), lambda g: (0, 0)),
            pl.BlockSpec((3, 3), lambda g: (0, 0)),
            pl.BlockSpec((1, FW), lambda g: (0, 0)),
            pl.BlockSpec((1, 3), lambda g: (0, 0)),
        ],
        out_specs=[
            pl.BlockSpec((RB, cout), lambda g: (g, 0)),
            pl.BlockSpec((cout, cout), lambda g: (0, 0)),
            pl.BlockSpec((1, cout), lambda g: (0, 0)),
        ],
        out_shape=[
            jax.ShapeDtypeStruct((P, cout), jnp.float32),
            jax.ShapeDtypeStruct((cout, cout), jnp.float32),
            jax.ShapeDtypeStruct((1, cout), jnp.float32),
        ],
        scratch_shapes=[
            pltpu.VMEM((cout, FW), jnp.float32),
            pltpu.VMEM((cout, 3), jnp.float32),
            pltpu.VMEM((1, cout), jnp.float32),
            pltpu.VMEM((cout, cout), jnp.float32),
            pltpu.VMEM((1, cout), jnp.float32),
        ],
    )(x, cxyz, vf, vc, brow, grow, berow, gff, gfc, gcc, sf, sc_)


def _eye(n):
    return (jax.lax.broadcasted_iota(jnp.int32, (n, n), 0)
            == jax.lax.broadcasted_iota(jnp.int32, (n, n), 1)).astype(jnp.float32)


def _mlp_body(cin, cout, last, x_ref, v_ref, b_ref, gm_ref, be_ref,
              gp_ref, sp_ref, *rest):
    if last:
        z_ref, wf_s, brow_s = rest
    else:
        z_ref, g1_ref, s1_ref, wf_s, brow_s, ag1, as1 = rest
    g = pl.program_id(0)

    @pl.when(g == 0)
    def _():
        v = v_ref[...]
        gp = gp_ref[...]
        sp = sp_ref[...]
        invp = jnp.float32(1.0 / P)
        ml = lax.dot_general(sp, v, (((1,), (1,)), ((), ())),
                             preferred_element_type=jnp.float32) * invp
        t1 = lax.dot_general(v, gp, (((1,), (0,)), ((), ())),
                             preferred_element_type=jnp.float32)
        q = jnp.sum(t1 * v, axis=1, keepdims=True)
        eye = _eye(cout)
        qrow = lax.dot_general(q, eye, (((0,), (0,)), ((), ())),
                               preferred_element_type=jnp.float32)
        var = qrow * invp - ml * ml
        a = gm_ref[...] / jnp.sqrt(var + 1e-5)
        mean = ml + b_ref[...]
        brow_s[...] = a * (b_ref[...] - mean) + be_ref[...]
        acol = lax.dot_general(eye, a, (((1,), (1,)), ((), ())),
                               preferred_element_type=jnp.float32)
        wf_s[...] = v * acol

    x = x_ref[...]
    y = lax.dot_general(x, wf_s[...], (((1,), (1,)), ((), ())),
                        preferred_element_type=jnp.float32) + brow_s[...]
    z = jnp.maximum(y, 0.0)

    if last:
        z_ref[...] = jnp.max(z.reshape(GB, NSAMPLE, cout), axis=1)
    else:
        z_ref[...] = z
        g1c = lax.dot_general(z, z, (((0,), (0,)), ((), ())),
                              preferred_element_type=jnp.float32)
        s1c = jnp.sum(z, axis=0, keepdims=True)

        @pl.when(g == 0)
        def _():
            ag1[...] = jnp.zeros_like(ag1)
            as1[...] = jnp.zeros_like(as1)

        ag1[...] += g1c
        as1[...] += s1c

        @pl.when(g == NBLK - 1)
        def _():
            g1_ref[...] = ag1[...]
            s1_ref[...] = as1[...]


def _run_mlp_layer(x, v, brow, grow, berow, gp, sp, last=False):
    cin = x.shape[1]
    cout = v.shape[0]
    small = [
        pl.BlockSpec((cout, cin), lambda g: (0, 0)),
        pl.BlockSpec((1, cout), lambda g: (0, 0)),
        pl.BlockSpec((1, cout), lambda g: (0, 0)),
        pl.BlockSpec((1, cout), lambda g: (0, 0)),
        pl.BlockSpec((cin, cin), lambda g: (0, 0)),
        pl.BlockSpec((1, cin), lambda g: (0, 0)),
    ]
    if last:
        out_specs = pl.BlockSpec((GB, cout), lambda g: (g, 0))
        out_shape = jax.ShapeDtypeStruct((NROWS, cout), jnp.float32)
        scratch = [pltpu.VMEM((cout, cin), jnp.float32),
                   pltpu.VMEM((1, cout), jnp.float32)]
    else:
        out_specs = [
            pl.BlockSpec((RB, cout), lambda g: (g, 0)),
            pl.BlockSpec((cout, cout), lambda g: (0, 0)),
            pl.BlockSpec((1, cout), lambda g: (0, 0)),
        ]
        out_shape = [
            jax.ShapeDtypeStruct((P, cout), jnp.float32),
            jax.ShapeDtypeStruct((cout, cout), jnp.float32),
            jax.ShapeDtypeStruct((1, cout), jnp.float32),
        ]
        scratch = [pltpu.VMEM((cout, cin), jnp.float32),
                   pltpu.VMEM((1, cout), jnp.float32),
                   pltpu.VMEM((cout, cout), jnp.float32),
                   pltpu.VMEM((1, cout), jnp.float32)]

    def body(*refs):
        _mlp_body(cin, cout, last, *refs)

    return pl.pallas_call(
        body,
        grid=(NBLK,),
        in_specs=[pl.BlockSpec((RB, cin), lambda g: (g, 0))] + small,
        out_specs=out_specs,
        out_shape=out_shape,
        scratch_shapes=scratch,
    )(x, v, brow, grow, berow, gp, sp)


def kernel(xyz, points, seed_inds, W0, b0, gamma0, beta0,
           W1, b1, gamma1, beta1, W2, b2, gamma2, beta2):
    fps_idx, nx, ny, nz = _run_fps(xyz)
    new_xyz_t = jnp.stack([nx, ny, nz], axis=-1)
    seeds32 = seed_inds.astype(jnp.int32)
    seeds_lo = (seeds32 % 256).astype(jnp.float32).reshape(B, 1, N)
    seeds_hi = (seeds32 // 256).astype(jnp.float32).reshape(B, 1, N)
    feat3 = jnp.concatenate(
        [xyz.transpose(0, 2, 1), points.transpose(0, 2, 1),
         jnp.zeros((B, N, FW - 67), jnp.float32)], axis=-1)
    seed_g, idx_out, Gff, Gfc, Gcc, sf, sc_ = _run_mask(
        new_xyz_t, xyz, fps_idx.reshape(B, NPOINT, 1),
        seeds_lo, seeds_hi, feat3)

    X = _run_sc(idx_out.reshape(-1), feat3.reshape(B * N, FW))
    cxyz = new_xyz_t.reshape(NROWS, 3)

    co0 = W0.shape[0]
    Vf = jnp.concatenate(
        [W0, jnp.zeros((co0, FW - 67), jnp.float32)], axis=1)
    Vc = -W0[:, :3]
    z0, G1, s1 = _run_mlp0(X, cxyz, Vf, Vc, b0.reshape(1, -1),
                           gamma0.reshape(1, -1), beta0.reshape(1, -1),
                           Gff, Gfc, Gcc, sf, sc_)
    z1, G2, s2 = _run_mlp_layer(z0, W1, b1.reshape(1, -1),
                                gamma1.reshape(1, -1), beta1.reshape(1, -1),
                                G1, s1)
    Y = _run_mlp_layer(z1, W2, b2.reshape(1, -1),
                       gamma2.reshape(1, -1), beta2.reshape(1, -1),
                       G2, s2, last=True)

    new_points_out = Y.reshape(B, NPOINT, W2.shape[0]).transpose(0, 2, 1)
    new_xyz_out = jnp.stack([nx, ny, nz], axis=1)
    this_seed = seed_g.reshape(B, NPOINT).astype(seed_inds.dtype)
    return (new_xyz_out, new_points_out, this_seed)


# speedup vs baseline: 1.1238x; 1.0706x over previous
import jax
import jax.numpy as jnp
from jax import lax
from jax.experimental import pallas as pl
from jax.experimental.pallas import tpu as pltpu
from jax.experimental.pallas import tpu_sc as plsc

B, N, DPTS = 8, 2048, 64
NPOINT, RADIUS, NSAMPLE = 512, 0.2, 32
FW = 128
NROWS = B * NPOINT
P = NROWS * NSAMPLE
NW = 32
RPW = NROWS // NW
RB = 4096
GB = RB // NSAMPLE
NBLK = P // RB


NGF = 1
HB = B // NGF


def _fps_body(a_ref, idx_ref, cx_ref, cy_ref, cz_ref):
    a = a_ref[...]
    lane = jax.lax.broadcasted_iota(jnp.int32, (HB, N), 1)
    lane3 = jax.lax.broadcasted_iota(jnp.int32, (3 * HB, N), 1)
    out_lane = jax.lax.broadcasted_iota(jnp.int32, (HB, NPOINT), 1)
    slabs = [a[3 * HB * g:3 * HB * (g + 1)] for g in range(NGF)]

    def step_grp(slab, st, i):
        dist, far, cents, ox, oy, oz = st
        far3 = jnp.concatenate([far, far, far], axis=0)
        oh = lane3 == far3
        cvec = jnp.sum(jnp.where(oh, slab, 0.0), axis=1, keepdims=True)
        sel = out_lane == i
        cents = jnp.where(sel, far, cents)
        ox = jnp.where(sel, cvec[0:HB], ox)
        oy = jnp.where(sel, cvec[HB:2 * HB], oy)
        oz = jnp.where(sel, cvec[2 * HB:3 * HB], oz)
        df = slab - cvec
        sq = df * df
        d = sq[0:HB] + sq[HB:2 * HB]
        d = d + sq[2 * HB:3 * HB]
        dist = jnp.minimum(dist, d)
        m = jnp.max(dist, axis=1, keepdims=True)
        far = jnp.min(jnp.where(dist == m, lane, N), axis=1, keepdims=True)
        return (dist, far, cents, ox, oy, oz)

    def body(i, sts):
        return tuple(step_grp(slabs[g], sts[g], i) for g in range(NGF))

    def init(slab):
        dist0 = jnp.full((HB, N), 1e10, dtype=jnp.float32) + slab[0:HB] * 0.0
        far0 = jnp.min(lane, axis=1, keepdims=True)
        ox0 = slab[0:HB, :NPOINT] * 0.0
        return (dist0, far0, ox0.astype(jnp.int32), ox0, ox0, ox0)

    sts = jax.lax.fori_loop(0, NPOINT, body,
                            tuple(init(slabs[g]) for g in range(NGF)))
    idx_ref[...] = jnp.concatenate([st[2] for st in sts], axis=0)
    cx_ref[...] = jnp.concatenate([st[3] for st in sts], axis=0)
    cy_ref[...] = jnp.concatenate([st[4] for st in sts], axis=0)
    cz_ref[...] = jnp.concatenate([st[5] for st in sts], axis=0)


def _run_fps(xyz):
    a = xyz.reshape(NGF, HB, 3, N).transpose(0, 2, 1, 3).reshape(3 * B, N)
    out_shape = [
        jax.ShapeDtypeStruct((B, NPOINT), jnp.int32),
        jax.ShapeDtypeStruct((B, NPOINT), jnp.float32),
        jax.ShapeDtypeStruct((B, NPOINT), jnp.float32),
        jax.ShapeDtypeStruct((B, NPOINT), jnp.float32),
    ]
    return pl.pallas_call(_fps_body, out_shape=out_shape)(a)


NCH = N // 16


def _ball_cumsum(c, p):
    t = jax.lax.dot_general(c, p, (((1,), (0,)), ((), ())),
                            preferred_element_type=jnp.float32)
    d = -2.0 * t
    src2 = jnp.sum(c * c, axis=1, keepdims=True)
    d = d + src2
    dst2 = jnp.sum(p * p, axis=0, keepdims=True)
    d = d + dst2
    maskv = jnp.where(d <= RADIUS * RADIUS, 1.0, 0.0).astype(jnp.float32)

    lane = jax.lax.broadcasted_iota(jnp.int32, (NPOINT, N), 1)
    lmod = lane & 15
    cw = maskv
    for k in (1, 2, 4, 8):
        sh = jnp.concatenate(
            [jnp.zeros((NPOINT, k), jnp.float32), cw[:, :N - k]], axis=1)
        cw = cw + jnp.where(lmod >= k, sh, 0.0)
    cs_m = (jax.lax.broadcasted_iota(jnp.int32, (N, NCH), 0) // 16
            == jax.lax.broadcasted_iota(jnp.int32, (N, NCH), 1)
            ).astype(jnp.float32)
    c128 = jax.lax.dot_general(maskv, cs_m, (((1,), (0,)), ((), ())),
                               preferred_element_type=jnp.float32)
    lts = (jax.lax.broadcasted_iota(jnp.int32, (NCH, NCH), 0)
           < jax.lax.broadcasted_iota(jnp.int32, (NCH, NCH), 1)
           ).astype(jnp.float32)
    coff = jax.lax.dot_general(c128, lts, (((1,), (0,)), ((), ())),
                               preferred_element_type=jnp.float32)
    chi = jnp.floor(coff * (1.0 / 256.0))
    clo = coff - chi * 256.0
    er = (jax.lax.broadcasted_iota(jnp.int32, (NCH, N), 0)
          == jax.lax.broadcasted_iota(jnp.int32, (NCH, N), 1) // 16
          ).astype(jnp.float32)
    exp = (jax.lax.dot_general(chi, er, (((1,), (0,)), ((), ())),
                               preferred_element_type=jnp.float32) * 256.0
           + jax.lax.dot_general(clo, er, (((1,), (0,)), ((), ())),
                                 preferred_element_type=jnp.float32))
    return maskv, cw + exp


def _sel_body(c_ref, p_ref, idx_ref):
    b = pl.program_id(0)
    maskv, cincl = _ball_cumsum(c_ref[0], p_ref[0])

    out32 = jax.lax.broadcasted_iota(jnp.int32, (NPOINT, NSAMPLE), 1)

    def body(k, r):
        kf = jnp.float32(1.0) * k
        rk = jnp.sum(jnp.where(cincl <= kf, 1.0, 0.0), axis=1, keepdims=True)
        return jnp.where(out32 == k, rk, r)

    r0 = maskv[:, :NSAMPLE] * 0.0
    r = jax.lax.fori_loop(0, NSAMPLE, body, r0)
    first = jnp.sum(jnp.where(out32 == 0, r, 0.0), axis=1, keepdims=True)
    idxf = jnp.where(r == jnp.float32(N), first, r)
    idxf = idxf + jnp.float32(1.0) * (b * N)
    idx_ref[0] = idxf.astype(jnp.int32)


def _run_sel(new_xyz_t, xyz):
    return pl.pallas_call(
        _sel_body,
        grid=(B,),
        in_specs=[
            pl.BlockSpec((1, NPOINT, 3), lambda b: (b, 0, 0)),
            pl.BlockSpec((1, 3, N), lambda b: (b, 0, 0)),
        ],
        out_specs=pl.BlockSpec((1, NPOINT, NSAMPLE), lambda b: (b, 0, 0)),
        out_shape=jax.ShapeDtypeStruct((B, NPOINT, NSAMPLE), jnp.int32),
    )(new_xyz_t, xyz)


def _mask_body(c_ref, p_ref, f_ref, slo_ref, shi_ref, ft_ref,
               sg_ref, gff_ref, gfc_ref, gcc_ref, sf_ref, sc_ref,
               aff, afc, acc_, asf, asc):
    b = pl.program_id(0)
    c = c_ref[0]
    maskv, cincl = _ball_cumsum(c, p_ref[0])

    oh = (jax.lax.broadcasted_iota(jnp.int32, (NPOINT, N), 1)
          == f_ref[0]).astype(jnp.float32)
    lo = jax.lax.dot_general(slo_ref[0], oh, (((1,), (1,)), ((), ())),
                             preferred_element_type=jnp.float32)
    hi = jax.lax.dot_general(shi_ref[0], oh, (((1,), (1,)), ((), ())),
                             preferred_element_type=jnp.float32)
    sg_ref[0] = hi.astype(jnp.int32) * 256 + lo.astype(jnp.int32)

    ft = ft_ref[0]
    cmat = c
    selmat = maskv * jnp.where(cincl <= jnp.float32(NSAMPLE), 1.0, 0.0)
    tot = cincl[:, N - 1:N]
    padcnt = jnp.maximum(jnp.float32(NSAMPLE) - tot, 0.0)
    firstoh = maskv * jnp.where(cincl == 1.0, 1.0, 0.0)
    wmat = selmat + firstoh * padcnt
    onesc = jnp.zeros((NPOINT, 1), jnp.float32) + 1.0
    multc = jax.lax.dot_general(wmat, onesc, (((0,), (0,)), ((), ())),
                                preferred_element_type=jnp.float32)
    gffc = jax.lax.dot_general(ft * multc, ft, (((0,), (0,)), ((), ())),
                               preferred_element_type=jnp.float32)
    fs = jax.lax.dot_general(wmat, ft, (((1,), (0,)), ((), ())),
                             preferred_element_type=jnp.float32)
    gfcc = jax.lax.dot_general(fs, cmat, (((0,), (0,)), ((), ())),
                               preferred_element_type=jnp.float32)
    gccc = jax.lax.dot_general(cmat, cmat, (((0,), (0,)), ((), ())),
                               preferred_element_type=jnp.float32) * \
        jnp.float32(NSAMPLE)
    sfc = jnp.sum(fs, axis=0, keepdims=True)
    scc = jnp.sum(cmat, axis=0, keepdims=True) * jnp.float32(NSAMPLE)

    @pl.when(b == 0)
    def _():
        aff[...] = jnp.zeros_like(aff)
        afc[...] = jnp.zeros_like(afc)
        acc_[...] = jnp.zeros_like(acc_)
        asf[...] = jnp.zeros_like(asf)
        asc[...] = jnp.zeros_like(asc)

    aff[...] += gffc
    afc[...] += gfcc
    acc_[...] += gccc
    asf[...] += sfc
    asc[...] += scc

    @pl.when(b == B - 1)
    def _():
        gff_ref[...] = aff[...]
        gfc_ref[...] = afc[...]
        gcc_ref[...] = acc_[...]
        sf_ref[...] = asf[...]
        sc_ref[...] = asc[...]


def _run_mask(new_xyz_t, xyz, fps3, seeds_lo, seeds_hi, feat3):
    return pl.pallas_call(
        _mask_body,
        grid=(B,),
        in_specs=[
            pl.BlockSpec((1, NPOINT, 3), lambda b: (b, 0, 0)),
            pl.BlockSpec((1, 3, N), lambda b: (b, 0, 0)),
            pl.BlockSpec((1, NPOINT, 1), lambda b: (b, 0, 0)),
            pl.BlockSpec((1, 1, N), lambda b: (b, 0, 0)),
            pl.BlockSpec((1, 1, N), lambda b: (b, 0, 0)),
            pl.BlockSpec((1, N, FW), lambda b: (b, 0, 0)),
        ],
        out_specs=[
            pl.BlockSpec((1, 1, NPOINT), lambda b: (b, 0, 0)),
            pl.BlockSpec((FW, FW), lambda b: (0, 0)),
            pl.BlockSpec((FW, 3), lambda b: (0, 0)),
            pl.BlockSpec((3, 3), lambda b: (0, 0)),
            pl.BlockSpec((1, FW), lambda b: (0, 0)),
            pl.BlockSpec((1, 3), lambda b: (0, 0)),
        ],
        out_shape=[
            jax.ShapeDtypeStruct((B, 1, NPOINT), jnp.int32),
            jax.ShapeDtypeStruct((FW, FW), jnp.float32),
            jax.ShapeDtypeStruct((FW, 3), jnp.float32),
            jax.ShapeDtypeStruct((3, 3), jnp.float32),
            jax.ShapeDtypeStruct((1, FW), jnp.float32),
            jax.ShapeDtypeStruct((1, 3), jnp.float32),
        ],
        scratch_shapes=[
            pltpu.VMEM((FW, FW), jnp.float32),
            pltpu.VMEM((FW, 3), jnp.float32),
            pltpu.VMEM((3, 3), jnp.float32),
            pltpu.VMEM((1, FW), jnp.float32),
            pltpu.VMEM((1, 3), jnp.float32),
        ],
    )(new_xyz_t, xyz, fps3, seeds_lo, seeds_hi, feat3)


GROUP = 128
NGRP = RPW * NSAMPLE // GROUP


def _sc_body(idx_hbm, feat_hbm, x_hbm, ib0, ib1, rb0, rb1, sem0, sem1):
    cid = lax.axis_index("c")
    sid = lax.axis_index("s")
    w = sid * 2 + cid
    base = w * RPW * NSAMPLE

    pltpu.sync_copy(idx_hbm.at[pl.ds(base, GROUP)], ib0)
    pltpu.async_copy(feat_hbm.at[ib0], rb0, sem0)

    def g_it(h, carry):
        g0 = 2 * h
        pltpu.sync_copy(idx_hbm.at[pl.ds(base + (g0 + 1) * GROUP, GROUP)], ib1)
        pltpu.async_copy(feat_hbm.at[ib1], rb1, sem1)
        pltpu.make_async_copy(feat_hbm.at[ib0], rb0, sem0).wait()
        pltpu.sync_copy(rb0, x_hbm.at[pl.ds(base + g0 * GROUP, GROUP)])

        @pl.when(g0 + 2 < NGRP)
        def _():
            pltpu.sync_copy(idx_hbm.at[pl.ds(base + (g0 + 2) * GROUP, GROUP)],
                            ib0)
            pltpu.async_copy(feat_hbm.at[ib0], rb0, sem0)

        pltpu.make_async_copy(feat_hbm.at[ib1], rb1, sem1).wait()
        pltpu.sync_copy(rb1, x_hbm.at[pl.ds(base + (g0 + 1) * GROUP, GROUP)])
        return carry

    lax.fori_loop(0, NGRP // 2, g_it, 0)


def _run_sc(idxflat, feat):
    mesh = plsc.VectorSubcoreMesh(core_axis_name="c", subcore_axis_name="s")
    f = pl.kernel(
        _sc_body,
        out_type=jax.ShapeDtypeStruct((P, FW), jnp.float32),
        mesh=mesh,
        scratch_types=[
            pltpu.VMEM((GROUP,), jnp.int32),
            pltpu.VMEM((GROUP,), jnp.int32),
            pltpu.VMEM((GROUP, FW), jnp.float32),
            pltpu.VMEM((GROUP, FW), jnp.float32),
            pltpu.SemaphoreType.DMA,
            pltpu.SemaphoreType.DMA,
        ],
    )
    return f(idxflat, feat)


def _expand_mat():
    return (jax.lax.broadcasted_iota(jnp.int32, (RB, GB), 0) // NSAMPLE
            == jax.lax.broadcasted_iota(jnp.int32, (RB, GB), 1)
            ).astype(jnp.float32)


def _m1_body(x_ref, c_ref, gff_ref, gfc_ref, gcc_ref, sf_ref, sc_ref,
             aff, afc, acc_, asf, asc):
    g = pl.program_id(0)
    x = x_ref[...]
    ce = jax.lax.dot_general(_expand_mat(), c_ref[...], (((1,), (0,)), ((), ())),
                             preferred_element_type=jnp.float32)
    gffc = jax.lax.dot_general(x, x, (((0,), (0,)), ((), ())),
                               preferred_element_type=jnp.float32)
    gfcc = jax.lax.dot_general(x, ce, (((0,), (0,)), ((), ())),
                               preferred_element_type=jnp.float32)
    gccc = jax.lax.dot_general(ce, ce, (((0,), (0,)), ((), ())),
                               preferred_element_type=jnp.float32)
    sfc = jnp.sum(x, axis=0, keepdims=True)
    scc = jnp.sum(ce, axis=0, keepdims=True)

    @pl.when(g == 0)
    def _():
        aff[...] = jnp.zeros_like(aff)
        afc[...] = jnp.zeros_like(afc)
        acc_[...] = jnp.zeros_like(acc_)
        asf[...] = jnp.zeros_like(asf)
        asc[...] = jnp.zeros_like(asc)

    aff[...] += gffc
    afc[...] += gfcc
    acc_[...] += gccc
    asf[...] += sfc
    asc[...] += scc

    @pl.when(g == NBLK - 1)
    def _():
        gff_ref[...] = aff[...]
        gfc_ref[...] = afc[...]
        gcc_ref[...] = acc_[...]
        sf_ref[...] = asf[...]
        sc_ref[...] = asc[...]


def _run_m1(X, cxyz):
    return pl.pallas_call(
        _m1_body,
        grid=(NBLK,),
        in_specs=[
            pl.BlockSpec((RB, FW), lambda g: (g, 0)),
            pl.BlockSpec((GB, 3), lambda g: (g, 0)),
        ],
        out_specs=[
            pl.BlockSpec((FW, FW), lambda g: (0, 0)),
            pl.BlockSpec((FW, 3), lambda g: (0, 0)),
            pl.BlockSpec((3, 3), lambda g: (0, 0)),
            pl.BlockSpec((1, FW), lambda g: (0, 0)),
            pl.BlockSpec((1, 3), lambda g: (0, 0)),
        ],
        out_shape=[
            jax.ShapeDtypeStruct((FW, FW), jnp.float32),
            jax.ShapeDtypeStruct((FW, 3), jnp.float32),
            jax.ShapeDtypeStruct((3, 3), jnp.float32),
            jax.ShapeDtypeStruct((1, FW), jnp.float32),
            jax.ShapeDtypeStruct((1, 3), jnp.float32),
        ],
        scratch_shapes=[
            pltpu.VMEM((FW, FW), jnp.float32),
            pltpu.VMEM((FW, 3), jnp.float32),
            pltpu.VMEM((3, 3), jnp.float32),
            pltpu.VMEM((1, FW), jnp.float32),
            pltpu.VMEM((1, 3), jnp.float32),
        ],
    )(X, cxyz)


def _mlp0_body(x_ref, c_ref, vf_ref, vc_ref, b_ref, gm_ref, be_ref,
               gff_ref, gfc_ref, gcc_ref, sf_ref, sc_ref,
               z_ref, g1_ref, s1_ref, wf_s, wc_s, brow_s, ag1, as1):
    g = pl.program_id(0)
    cout = vf_ref.shape[0]

    @pl.when(g == 0)
    def _():
        vf = vf_ref[...]
        vc = vc_ref[...]
        invp = jnp.float32(1.0 / P)
        ml = (lax.dot_general(sf_ref[...], vf, (((1,), (1,)), ((), ())),
                              preferred_element_type=jnp.float32)
              + lax.dot_general(sc_ref[...], vc, (((1,), (1,)), ((), ())),
                                preferred_element_type=jnp.float32)) * invp
        t1 = lax.dot_general(vf, gff_ref[...], (((1,), (0,)), ((), ())),
                             preferred_element_type=jnp.float32)
        q = jnp.sum(t1 * vf, axis=1, keepdims=True)
        t2 = lax.dot_general(vf, gfc_ref[...], (((1,), (0,)), ((), ())),
                             preferred_element_type=jnp.float32)
        q = q + 2.0 * jnp.sum(t2 * vc, axis=1, keepdims=True)
        t3 = lax.dot_general(vc, gcc_ref[...], (((1,), (0,)), ((), ())),
                             preferred_element_type=jnp.float32)
        q = q + jnp.sum(t3 * vc, axis=1, keepdims=True)
        eye = _eye(cout)
        qrow = lax.dot_general(q, eye, (((0,), (0,)), ((), ())),
                               preferred_element_type=jnp.float32)
        var = qrow * invp - ml * ml
        a = gm_ref[...] / jnp.sqrt(var + 1e-5)
        brow_s[...] = a * (-ml) + be_ref[...]
        acol = lax.dot_general(eye, a, (((1,), (1,)), ((), ())),
                               preferred_element_type=jnp.float32)
        wf_s[...] = vf_ref[...] * acol
        wc_s[...] = vc_ref[...] * acol

    x = x_ref[...]
    ce = jax.lax.dot_general(_expand_mat(), c_ref[...], (((1,), (0,)), ((), ())),
                             preferred_element_type=jnp.float32)
    y = (lax.dot_general(x, wf_s[...], (((1,), (1,)), ((), ())),
                         preferred_element_type=jnp.float32)
         + lax.dot_general(ce, wc_s[...], (((1,), (1,)), ((), ())),
                           preferred_element_type=jnp.float32)
         + brow_s[...])
    z = jnp.maximum(y, 0.0)
    z_ref[...] = z
    g1c = lax.dot_general(z, z, (((0,), (0,)), ((), ())),
                          preferred_element_type=jnp.float32)
    s1c = jnp.sum(z, axis=0, keepdims=True)

    @pl.when(g == 0)
    def _():
        ag1[...] = jnp.zeros_like(ag1)
        as1[...] = jnp.zeros_like(as1)

    ag1[...] += g1c
    as1[...] += s1c

    @pl.when(g == NBLK - 1)
    def _():
        g1_ref[...] = ag1[...]
        s1_ref[...] = as1[...]


def _run_mlp0(x, cxyz, vf, vc, brow, grow, berow, gff, gfc, gcc, sf, sc_):
    cout = vf.shape[0]
    return pl.pallas_call(
        _mlp0_body,
        grid=(NBLK,),
        in_specs=[
            pl.BlockSpec((RB, FW), lambda g: (g, 0)),
            pl.BlockSpec((GB, 3), lambda g: (g, 0)),
            pl.BlockSpec((cout, FW), lambda g: (0, 0)),
            pl.BlockSpec((cout, 3), lambda g: (0, 0)),
            pl.BlockSpec((1, cout), lambda g: (0, 0)),
            pl.BlockSpec((1, cout), lambda g: (0, 0)),
            pl.BlockSpec((1, cout), lambda g: (0, 0)),
            pl.BlockSpec((FW, FW), lambda g: (0, 0)),
            pl.BlockSpec((FW, 3), lambda g: (0, 0)),
            pl.BlockSpec((3, 3), lambda g: (0, 0)),
            pl.BlockSpec((1, FW), lambda g: (0, 0)),
            pl.BlockSpec((1, 3), lambda g: (0, 0)),
        ],
        out_specs=[
            pl.BlockSpec((RB, cout), lambda g: (g, 0)),
            pl.BlockSpec((cout, cout), lambda g: (0, 0)),
            pl.BlockSpec((1, cout), lambda g: (0, 0)),
        ],
        out_shape=[
            jax.ShapeDtypeStruct((P, cout), jnp.float32),
            jax.ShapeDtypeStruct((cout, cout), jnp.float32),
            jax.ShapeDtypeStruct((1, cout), jnp.float32),
        ],
        scratch_shapes=[
            pltpu.VMEM((cout, FW), jnp.float32),
            pltpu.VMEM((cout, 3), jnp.float32),
            pltpu.VMEM((1, cout), jnp.float32),
            pltpu.VMEM((cout, cout), jnp.float32),
            pltpu.VMEM((1, cout), jnp.float32),
        ],
    )(x, cxyz, vf, vc, brow, grow, berow, gff, gfc, gcc, sf, sc_)


def _eye(n):
    return (jax.lax.broadcasted_iota(jnp.int32, (n, n), 0)
            == jax.lax.broadcasted_iota(jnp.int32, (n, n), 1)).astype(jnp.float32)


def _mlp_body(cin, cout, last, x_ref, v_ref, b_ref, gm_ref, be_ref,
              gp_ref, sp_ref, *rest):
    if last:
        z_ref, wf_s, brow_s = rest
    else:
        z_ref, g1_ref, s1_ref, wf_s, brow_s, ag1, as1 = rest
    g = pl.program_id(0)

    @pl.when(g == 0)
    def _():
        v = v_ref[...]
        gp = gp_ref[...]
        sp = sp_ref[...]
        invp = jnp.float32(1.0 / P)
        ml = lax.dot_general(sp, v, (((1,), (1,)), ((), ())),
                             preferred_element_type=jnp.float32) * invp
        t1 = lax.dot_general(v, gp, (((1,), (0,)), ((), ())),
                             preferred_element_type=jnp.float32)
        q = jnp.sum(t1 * v, axis=1, keepdims=True)
        eye = _eye(cout)
        qrow = lax.dot_general(q, eye, (((0,), (0,)), ((), ())),
                               preferred_element_type=jnp.float32)
        var = qrow * invp - ml * ml
        a = gm_ref[...] / jnp.sqrt(var + 1e-5)
        mean = ml + b_ref[...]
        brow_s[...] = a * (b_ref[...] - mean) + be_ref[...]
        acol = lax.dot_general(eye, a, (((1,), (1,)), ((), ())),
                               preferred_element_type=jnp.float32)
        wf_s[...] = v * acol

    x = x_ref[...]
    y = lax.dot_general(x, wf_s[...], (((1,), (1,)), ((), ())),
                        preferred_element_type=jnp.float32) + brow_s[...]
    z = jnp.maximum(y, 0.0)

    if last:
        z_ref[...] = jnp.max(z.reshape(GB, NSAMPLE, cout), axis=1)
    else:
        z_ref[...] = z
        g1c = lax.dot_general(z, z, (((0,), (0,)), ((), ())),
                              preferred_element_type=jnp.float32)
        s1c = jnp.sum(z, axis=0, keepdims=True)

        @pl.when(g == 0)
        def _():
            ag1[...] = jnp.zeros_like(ag1)
            as1[...] = jnp.zeros_like(as1)

        ag1[...] += g1c
        as1[...] += s1c

        @pl.when(g == NBLK - 1)
        def _():
            g1_ref[...] = ag1[...]
            s1_ref[...] = as1[...]


def _run_mlp_layer(x, v, brow, grow, berow, gp, sp, last=False):
    cin = x.shape[1]
    cout = v.shape[0]
    small = [
        pl.BlockSpec((cout, cin), lambda g: (0, 0)),
        pl.BlockSpec((1, cout), lambda g: (0, 0)),
        pl.BlockSpec((1, cout), lambda g: (0, 0)),
        pl.BlockSpec((1, cout), lambda g: (0, 0)),
        pl.BlockSpec((cin, cin), lambda g: (0, 0)),
        pl.BlockSpec((1, cin), lambda g: (0, 0)),
    ]
    if last:
        out_specs = pl.BlockSpec((GB, cout), lambda g: (g, 0))
        out_shape = jax.ShapeDtypeStruct((NROWS, cout), jnp.float32)
        scratch = [pltpu.VMEM((cout, cin), jnp.float32),
                   pltpu.VMEM((1, cout), jnp.float32)]
    else:
        out_specs = [
            pl.BlockSpec((RB, cout), lambda g: (g, 0)),
            pl.BlockSpec((cout, cout), lambda g: (0, 0)),
            pl.BlockSpec((1, cout), lambda g: (0, 0)),
        ]
        out_shape = [
            jax.ShapeDtypeStruct((P, cout), jnp.float32),
            jax.ShapeDtypeStruct((cout, cout), jnp.float32),
            jax.ShapeDtypeStruct((1, cout), jnp.float32),
        ]
        scratch = [pltpu.VMEM((cout, cin), jnp.float32),
                   pltpu.VMEM((1, cout), jnp.float32),
                   pltpu.VMEM((cout, cout), jnp.float32),
                   pltpu.VMEM((1, cout), jnp.float32)]

    def body(*refs):
        _mlp_body(cin, cout, last, *refs)

    return pl.pallas_call(
        body,
        grid=(NBLK,),
        in_specs=[pl.BlockSpec((RB, cin), lambda g: (g, 0))] + small,
        out_specs=out_specs,
        out_shape=out_shape,
        scratch_shapes=scratch,
    )(x, v, brow, grow, berow, gp, sp)


def kernel(xyz, points, seed_inds, W0, b0, gamma0, beta0,
           W1, b1, gamma1, beta1, W2, b2, gamma2, beta2):
    fps_idx, nx, ny, nz = _run_fps(xyz)
    new_xyz_t = jnp.stack([nx, ny, nz], axis=-1)
    seeds32 = seed_inds.astype(jnp.int32)
    seeds_lo = (seeds32 % 256).astype(jnp.float32).reshape(B, 1, N)
    seeds_hi = (seeds32 // 256).astype(jnp.float32).reshape(B, 1, N)
    feat3 = jnp.concatenate(
        [xyz.transpose(0, 2, 1), points.transpose(0, 2, 1),
         jnp.zeros((B, N, FW - 67), jnp.float32)], axis=-1)
    idx_out = _run_sel(new_xyz_t, xyz)
    X = _run_sc(idx_out.reshape(-1), feat3.reshape(B * N, FW))
    seed_g, Gff, Gfc, Gcc, sf, sc_ = _run_mask(
        new_xyz_t, xyz, fps_idx.reshape(B, NPOINT, 1),
        seeds_lo, seeds_hi, feat3)
    cxyz = new_xyz_t.reshape(NROWS, 3)

    co0 = W0.shape[0]
    Vf = jnp.concatenate(
        [W0, jnp.zeros((co0, FW - 67), jnp.float32)], axis=1)
    Vc = -W0[:, :3]
    z0, G1, s1 = _run_mlp0(X, cxyz, Vf, Vc, b0.reshape(1, -1),
                           gamma0.reshape(1, -1), beta0.reshape(1, -1),
                           Gff, Gfc, Gcc, sf, sc_)
    z1, G2, s2 = _run_mlp_layer(z0, W1, b1.reshape(1, -1),
                                gamma1.reshape(1, -1), beta1.reshape(1, -1),
                                G1, s1)
    Y = _run_mlp_layer(z1, W2, b2.reshape(1, -1),
                       gamma2.reshape(1, -1), beta2.reshape(1, -1),
                       G2, s2, last=True)

    new_points_out = Y.reshape(B, NPOINT, W2.shape[0]).transpose(0, 2, 1)
    new_xyz_out = jnp.stack([nx, ny, nz], axis=1)
    this_seed = seed_g.reshape(B, NPOINT).astype(seed_inds.dtype)
    return (new_xyz_out, new_points_out, this_seed)
